# baseline (speedup 1.0000x reference)
import functools

import jax
import jax.numpy as jnp
from jax import lax
from jax.experimental import pallas as pl
from jax.experimental.pallas import tpu as pltpu
from jax.experimental.pallas import tpu_sc as plsc

B, S, H = 2, 2048, 1024
NH, NKV, HD = 16, 4, 64
I, E, K = 512, 8, 2
GRP = NH // NKV
EPS = 1e-6
T = B * S

BLK_T = 512
BLK_Q = 512
BLK_K = 512
QROWS = GRP * BLK_Q
NEG = -1e30

CAP = T
NBLK = 2 * T // BLK_T + E - 1
NW = 32
TPW = T // NW
HCH = TPW // 2


def _rms(x, w):
    var = jnp.mean(x * x, axis=-1, keepdims=True)
    return x * jax.lax.rsqrt(var + EPS) * w


def _pre_kernel(hs_ref, ln1_ref, wq_ref, wk_ref, wv_ref, q_ref, k_ref, v_ref):
    x = hs_ref[...]
    xn = _rms(x, ln1_ref[...]).astype(jnp.bfloat16)
    q = jax.lax.dot(xn, wq_ref[...],
                    preferred_element_type=jnp.float32).astype(jnp.bfloat16)
    k = jax.lax.dot(xn, wk_ref[...],
                    preferred_element_type=jnp.float32).astype(jnp.bfloat16)
    v = jax.lax.dot(xn, wv_ref[...],
                    preferred_element_type=jnp.float32).astype(jnp.bfloat16)
    for h in range(NH):
        q_ref[h] = q[:, h * HD:(h + 1) * HD]
    for h in range(NKV):
        k_ref[h] = k[:, h * HD:(h + 1) * HD]
        v_ref[h] = v[:, h * HD:(h + 1) * HD]


def _rope(x, cos, sin):
    x1 = x[:, :HD // 2]
    x2 = x[:, HD // 2:]
    rot = jnp.concatenate([-x2, x1], axis=-1)
    return x * cos + rot * sin


def _flash_kernel(iq_ref, ik_ref, q_ref, k_ref, v_ref, cq_ref, sq_ref,
                  ck_ref, sk_ref, mask_ref, o_ref, acc_ref, m_ref, l_ref):
    p_id = pl.program_id(2)
    iq = iq_ref[p_id]
    ik = ik_ref[p_id]

    @pl.when(ik == 0)
    def _():
        m_ref[...] = jnp.full_like(m_ref, NEG)
        l_ref[...] = jnp.zeros_like(l_ref)
        acc_ref[...] = jnp.zeros_like(acc_ref)

    q = _rope(q_ref[...].reshape(QROWS, HD).astype(jnp.float32),
              cq_ref[0], sq_ref[0])
    k = _rope(k_ref[0].astype(jnp.float32), ck_ref[...], sk_ref[...])
    qb = (q * (HD ** -0.5)).astype(jnp.bfloat16)
    kb = k.astype(jnp.bfloat16)
    s = jax.lax.dot_general(qb, kb, (((1,), (1,)), ((), ())),
                            preferred_element_type=jnp.float32)
    ind = (ik == iq).astype(jnp.float32)
    s = s + mask_ref[...] * ind

    m_prev = m_ref[...]
    l_prev = l_ref[...]
    m_cur = jnp.max(s, axis=1, keepdims=True)
    m_next = jnp.maximum(m_prev, m_cur)
    alpha = jnp.exp(m_prev - m_next)
    p = jnp.exp(s - m_next[:, :1])
    l_ref[...] = alpha * l_prev + jnp.sum(p, axis=1, keepdims=True)
    m_ref[...] = m_next
    pv = jax.lax.dot(p.astype(jnp.bfloat16), v_ref[0],
                     preferred_element_type=jnp.float32)
    acc_ref[...] = acc_ref[...] * alpha[:, :1] + pv

    @pl.when(ik == iq)
    def _():
        out = acc_ref[...] / l_ref[:, :1]
        o_ref[...] = out.reshape(GRP, BLK_Q, HD)


def _post_kernel(attn_ref, hs_ref, ln2_ref, wo_ref, rw_ref,
                 res2_ref, xn2_ref, wv_ref, pidx_ref, desc_ref, cnt_ref):
    i = pl.program_id(0)

    @pl.when(i == 0)
    def _():
        cnt_ref[...] = jnp.zeros_like(cnt_ref)

    a = jnp.concatenate([attn_ref[h] for h in range(NH)], axis=1)
    o = jax.lax.dot(a.astype(jnp.bfloat16), wo_ref[...],
                    preferred_element_type=jnp.float32)
    res2 = hs_ref[...] + o
    res2_ref[...] = res2
    xn = _rms(res2, ln2_ref[...])
    xnb = xn.astype(jnp.bfloat16)
    xn2_ref[...] = xnb.astype(jnp.float32)
    logits = jax.lax.dot(xnb, rw_ref[...], preferred_element_type=jnp.float32)
    lane = jax.lax.broadcasted_iota(jnp.int32, (BLK_T, 128), 1)
    valid = lane < E
    lg = jnp.where(valid, logits, NEG)
    mx = jnp.max(lg, axis=1, keepdims=True)
    ex = jnp.where(valid, jnp.exp(lg - mx), 0.0)
    denom = jnp.sum(ex, axis=1, keepdims=True)
    sc = ex / denom
    m1 = jnp.max(sc, axis=1, keepdims=True)
    idx1 = jnp.min(jnp.where(sc == m1, lane, 128), axis=1, keepdims=True)
    is1 = lane == idx1
    sc2 = jnp.where(is1, -1.0, sc)
    m2 = jnp.max(sc2, axis=1, keepdims=True)
    idx2 = jnp.min(jnp.where(sc2 == m2, lane, 128), axis=1, keepdims=True)
    is2 = lane == idx2
    wsum = m1 + m2
    wv_ref[...] = jnp.where(lane == 0, m1 / wsum,
                            jnp.where(lane == 1, m2 / wsum, 0.0))

    sel = (is1 | is2).astype(jnp.float32)
    rt = jax.lax.broadcasted_iota(jnp.int32, (BLK_T, BLK_T), 0)
    ct = jax.lax.broadcasted_iota(jnp.int32, (BLK_T, BLK_T), 1)
    tril = (ct < rt).astype(jnp.float32)
    prefix = jax.lax.dot(tril, sel, preferred_element_type=jnp.float32)
    dest = lane.astype(jnp.float32) * CAP + cnt_ref[...] + prefix
    p1f = jnp.sum(dest * is1.astype(jnp.float32), axis=1, keepdims=True)
    p2f = jnp.sum(dest * is2.astype(jnp.float32), axis=1, keepdims=True)
    pidx_ref[...] = jnp.where(lane == 0, p1f,
                              jnp.where(lane == 1, p2f, 0.0)).astype(jnp.int32)
    cnt_new = cnt_ref[...] + jnp.sum(sel, axis=0, keepdims=True)
    cnt_ref[...] = cnt_new

    @pl.when(i == T // BLK_T - 1)
    def _():
        rc = jax.lax.broadcasted_iota(jnp.int32, (128, 128), 0)
        cc = jax.lax.broadcasted_iota(jnp.int32, (128, 128), 1)
        amap = ((cc == rc // 8) & (rc < 64)).astype(jnp.float32)
        cntc = jax.lax.dot_general(amap, cnt_new, (((1,), (1,)), ((), ())),
                                   preferred_element_type=jnp.float32)
        ccol = jax.lax.broadcasted_iota(jnp.int32, (128, 1), 0)
        jcol = ccol % 8
        act = ((jcol.astype(jnp.float32) * BLK_T) < cntc) & (ccol < 64)
        actf = act.astype(jnp.float32)
        tri = (cc < rc).astype(jnp.float32)
        rank = jax.lax.dot(tri, actf, preferred_element_type=jnp.float32)
        srow = cc.astype(jnp.float32)
        smat = ((rank == srow) & act).astype(jnp.float32)
        ecolf = (ccol // 8).astype(jnp.float32)
        gcolf = ccol.astype(jnp.float32)
        blk_e = jax.lax.dot_general(ecolf, smat, (((0,), (0,)), ((), ())),
                                    preferred_element_type=jnp.float32)
        blk_g = jax.lax.dot_general(gcolf, smat, (((0,), (0,)), ((), ())),
                                    preferred_element_type=jnp.float32)
        zero6 = jnp.zeros((6, 128), jnp.float32)
        desc_ref[...] = jnp.concatenate([blk_e, blk_g, zero6],
                                        axis=0).astype(jnp.int32)


def _sc_mesh():
    return plsc.VectorSubcoreMesh(core_axis_name="c", subcore_axis_name="s")


def _sc_dispatch(xn2_3d, p1, p2):
    @functools.partial(
        pl.kernel,
        mesh=_sc_mesh(),
        out_type=jax.ShapeDtypeStruct((E * CAP, 8, 128), jnp.float32),
        scratch_types=[
            pltpu.VMEM((HCH,), jnp.int32),
            pltpu.VMEM((HCH,), jnp.int32),
            pltpu.VMEM((HCH, 8, 128), jnp.float32),
            pltpu.SemaphoreType.DMA,
            pltpu.SemaphoreType.DMA,
        ],
    )
    def k(xn2_hbm, p1_hbm, p2_hbm, xg_hbm, i1_v, i2_v, rows_v, sem1, sem2):
        wid = lax.axis_index("s") * 2 + lax.axis_index("c")
        base = wid * TPW
        for half in range(2):
            b2 = base + half * HCH
            pltpu.sync_copy(p1_hbm.at[pl.ds(b2, HCH)], i1_v)
            pltpu.sync_copy(p2_hbm.at[pl.ds(b2, HCH)], i2_v)
            pltpu.sync_copy(xn2_hbm.at[pl.ds(b2, HCH)], rows_v)
            c1 = pltpu.async_copy(rows_v, xg_hbm.at[i1_v], sem1)
            c2 = pltpu.async_copy(rows_v, xg_hbm.at[i2_v], sem2)
            c1.wait()
            c2.wait()

    return k(xn2_3d, p1, p2)


def _sc_collect(os_3d, p1, p2):
    @functools.partial(
        pl.kernel,
        mesh=_sc_mesh(),
        out_type=[
            jax.ShapeDtypeStruct((T, 8, 128), jnp.float32),
            jax.ShapeDtypeStruct((T, 8, 128), jnp.float32),
        ],
        scratch_types=[
            pltpu.VMEM((HCH,), jnp.int32),
            pltpu.VMEM((HCH, 8, 128), jnp.float32),
            pltpu.SemaphoreType.DMA,
        ],
    )
    def k(os_hbm, p1_hbm, p2_hbm, g1_hbm, g2_hbm, idx_v, rows_v, sem):
        wid = lax.axis_index("s") * 2 + lax.axis_index("c")
        base = wid * TPW
        for half in range(2):
            b2 = base + half * HCH
            pltpu.sync_copy(p1_hbm.at[pl.ds(b2, HCH)], idx_v)
            pltpu.async_copy(os_hbm.at[idx_v], rows_v, sem).wait()
            pltpu.sync_copy(rows_v, g1_hbm.at[pl.ds(b2, HCH)])
            pltpu.sync_copy(p2_hbm.at[pl.ds(b2, HCH)], idx_v)
            pltpu.async_copy(os_hbm.at[idx_v], rows_v, sem).wait()
            pltpu.sync_copy(rows_v, g2_hbm.at[pl.ds(b2, HCH)])

    return k(os_3d, p1, p2)


def _moe2_kernel(be_ref, bg_ref, x_ref, wg_ref, wu_ref, wd_ref, o_ref):
    x = x_ref[...].astype(jnp.bfloat16)
    gate = jax.lax.dot(x, wg_ref[0], preferred_element_type=jnp.float32)
    up = jax.lax.dot(x, wu_ref[0], preferred_element_type=jnp.float32)
    act = (gate * jax.lax.logistic(gate)) * up
    o_ref[...] = jax.lax.dot(act.astype(jnp.bfloat16), wd_ref[0],
                             preferred_element_type=jnp.float32)


def _comb_kernel(res2_ref, g1_ref, g2_ref, wv_ref, y_ref):
    wv = wv_ref[...]
    w1 = wv[:, :1]
    w2 = wv[:, 1:2]
    y_ref[...] = res2_ref[...] + w1 * g1_ref[...] + w2 * g2_ref[...]


def _build(hidden_states, cos, sin, Wq, Wk, Wv, Wo, ln1_w, ln2_w,
           router_w, Wg, Wu, Wd):
    hs = hidden_states.reshape(T, H)
    ln1 = ln1_w.reshape(1, H)
    ln2 = ln2_w.reshape(1, H)
    wq_t = Wq.T.astype(jnp.bfloat16)
    wk_t = Wk.T.astype(jnp.bfloat16)
    wv_t = Wv.T.astype(jnp.bfloat16)
    wo_t = Wo.T.astype(jnp.bfloat16)
    rw_pad = jnp.zeros((128, H), jnp.float32).at[:E].set(router_w)
    rw_t = rw_pad.T.astype(jnp.bfloat16)

    n_t = T // BLK_T
    q, k, v = pl.pallas_call(
        _pre_kernel,
        grid=(n_t,),
        in_specs=[
            pl.BlockSpec((BLK_T, H), lambda i: (i, 0)),
            pl.BlockSpec((1, H), lambda i: (0, 0)),
            pl.BlockSpec((H, NH * HD), lambda i: (0, 0)),
            pl.BlockSpec((H, NKV * HD), lambda i: (0, 0)),
            pl.BlockSpec((H, NKV * HD), lambda i: (0, 0)),
        ],
        out_specs=[
            pl.BlockSpec((NH, BLK_T, HD), lambda i: (0, i, 0)),
            pl.BlockSpec((NKV, BLK_T, HD), lambda i: (0, i, 0)),
            pl.BlockSpec((NKV, BLK_T, HD), lambda i: (0, i, 0)),
        ],
        out_shape=[
            jax.ShapeDtypeStruct((NH, T, HD), jnp.bfloat16),
            jax.ShapeDtypeStruct((NKV, T, HD), jnp.bfloat16),
            jax.ShapeDtypeStruct((NKV, T, HD), jnp.bfloat16),
        ],
    )(hs, ln1, wq_t, wk_t, wv_t)

    n_q = S // BLK_Q
    n_k = S // BLK_K
    cos_q = jnp.tile(cos.reshape(n_q, 1, BLK_Q, HD), (1, GRP, 1, 1)) \
        .reshape(n_q, QROWS, HD)
    sin_q = jnp.tile(sin.reshape(n_q, 1, BLK_Q, HD), (1, GRP, 1, 1)) \
        .reshape(n_q, QROWS, HD)
    r = jnp.arange(BLK_Q)[:, None]
    c = jnp.arange(BLK_K)[None, :]
    mask1 = jnp.where(r >= c, 0.0, NEG).astype(jnp.float32)
    mask = jnp.tile(mask1, (GRP, 1))

    pairs = [(a, b2) for a in range(n_q) for b2 in range(a + 1)]
    n_p = len(pairs)
    iq_arr = jnp.asarray([p[0] for p in pairs], jnp.int32)
    ik_arr = jnp.asarray([p[1] for p in pairs], jnp.int32)

    attn = pl.pallas_call(
        _flash_kernel,
        grid_spec=pltpu.PrefetchScalarGridSpec(
            num_scalar_prefetch=2,
            grid=(B, NKV, n_p),
            in_specs=[
                pl.BlockSpec((GRP, BLK_Q, HD),
                             lambda b, g, p, iqa, ika: (g, b * n_q + iqa[p], 0)),
                pl.BlockSpec((1, BLK_K, HD),
                             lambda b, g, p, iqa, ika: (g, b * n_k + ika[p], 0)),
                pl.BlockSpec((1, BLK_K, HD),
                             lambda b, g, p, iqa, ika: (g, b * n_k + ika[p], 0)),
                pl.BlockSpec((1, QROWS, HD),
                             lambda b, g, p, iqa, ika: (iqa[p], 0, 0)),
                pl.BlockSpec((1, QROWS, HD),
                             lambda b, g, p, iqa, ika: (iqa[p], 0, 0)),
                pl.BlockSpec((BLK_K, HD),
                             lambda b, g, p, iqa, ika: (ika[p], 0)),
                pl.BlockSpec((BLK_K, HD),
                             lambda b, g, p, iqa, ika: (ika[p], 0)),
                pl.BlockSpec((QROWS, BLK_K),
                             lambda b, g, p, iqa, ika: (0, 0)),
            ],
            out_specs=pl.BlockSpec(
                (GRP, BLK_Q, HD),
                lambda b, g, p, iqa, ika: (g, b * n_q + iqa[p], 0)),
            scratch_shapes=[
                pltpu.VMEM((QROWS, HD), jnp.float32),
                pltpu.VMEM((QROWS, 128), jnp.float32),
                pltpu.VMEM((QROWS, 128), jnp.float32),
            ],
        ),
        out_shape=jax.ShapeDtypeStruct((NH, T, HD), jnp.float32),
        compiler_params=pltpu.CompilerParams(
            dimension_semantics=("parallel", "parallel", "arbitrary"),
        ),
    )(iq_arr, ik_arr, q, k, v, cos_q, sin_q, cos, sin, mask)

    res2, xn2, wv, pidx, desc = pl.pallas_call(
        _post_kernel,
        grid=(n_t,),
        in_specs=[
            pl.BlockSpec((NH, BLK_T, HD), lambda i: (0, i, 0)),
            pl.BlockSpec((BLK_T, H), lambda i: (i, 0)),
            pl.BlockSpec((1, H), lambda i: (0, 0)),
            pl.BlockSpec((NH * HD, H), lambda i: (0, 0)),
            pl.BlockSpec((H, 128), lambda i: (0, 0)),
        ],
        out_specs=[
            pl.BlockSpec((BLK_T, H), lambda i: (i, 0)),
            pl.BlockSpec((BLK_T, H), lambda i: (i, 0)),
            pl.BlockSpec((BLK_T, 128), lambda i: (i, 0)),
            pl.BlockSpec((BLK_T, 128), lambda i: (i, 0)),
            pl.BlockSpec((8, 128), lambda i: (0, 0)),
        ],
        out_shape=[
            jax.ShapeDtypeStruct((T, H), jnp.float32),
            jax.ShapeDtypeStruct((T, H), jnp.float32),
            jax.ShapeDtypeStruct((T, 128), jnp.float32),
            jax.ShapeDtypeStruct((T, 128), jnp.int32),
            jax.ShapeDtypeStruct((8, 128), jnp.int32),
        ],
        scratch_shapes=[pltpu.VMEM((1, 128), jnp.float32)],
        compiler_params=pltpu.CompilerParams(
            dimension_semantics=("arbitrary",),
        ),
    )(attn, hs, ln2, wo_t, rw_t)

    p1 = pidx[:, 0]
    p2 = pidx[:, 1]
    be = desc[0]
    bg = desc[1]

    xg = _sc_dispatch(xn2.reshape(T, 8, 128), p1, p2)
    xg2 = xg.reshape(E * CAP, H)

    wg_t = Wg.transpose(0, 2, 1).astype(jnp.bfloat16)
    wu_t = Wu.transpose(0, 2, 1).astype(jnp.bfloat16)
    wd_t = Wd.transpose(0, 2, 1).astype(jnp.bfloat16)

    os_ = pl.pallas_call(
        _moe2_kernel,
        grid_spec=pltpu.PrefetchScalarGridSpec(
            num_scalar_prefetch=2,
            grid=(NBLK,),
            in_specs=[
                pl.BlockSpec((BLK_T, H), lambda s, be, bg: (bg[s], 0)),
                pl.BlockSpec((1, H, I), lambda s, be, bg: (be[s], 0, 0)),
                pl.BlockSpec((1, H, I), lambda s, be, bg: (be[s], 0, 0)),
                pl.BlockSpec((1, I, H), lambda s, be, bg: (be[s], 0, 0)),
            ],
            out_specs=pl.BlockSpec((BLK_T, H), lambda s, be, bg: (bg[s], 0)),
        ),
        out_shape=jax.ShapeDtypeStruct((E * CAP, H), jnp.float32),
        compiler_params=pltpu.CompilerParams(
            dimension_semantics=("arbitrary",),
        ),
    )(be, bg, xg2, wg_t, wu_t, wd_t)

    g1, g2 = _sc_collect(os_.reshape(E * CAP, 8, 128), p1, p2)

    y = pl.pallas_call(
        _comb_kernel,
        grid=(n_t,),
        in_specs=[
            pl.BlockSpec((BLK_T, H), lambda i: (i, 0)),
            pl.BlockSpec((BLK_T, H), lambda i: (i, 0)),
            pl.BlockSpec((BLK_T, H), lambda i: (i, 0)),
            pl.BlockSpec((BLK_T, 128), lambda i: (i, 0)),
        ],
        out_specs=pl.BlockSpec((BLK_T, H), lambda i: (i, 0)),
        out_shape=jax.ShapeDtypeStruct((T, H), jnp.float32),
    )(res2, g1.reshape(T, H), g2.reshape(T, H), wv)

    return y.reshape(B, S, H)


@jax.jit
def kernel(hidden_states, cos, sin, Wq, Wk, Wv, Wo, ln1_w, ln2_w,
           router_w, Wg, Wu, Wd):
    return _build(hidden_states, cos, sin, Wq, Wk, Wv, Wo, ln1_w, ln2_w,
                  router_w, Wg, Wu, Wd)


# speedup vs baseline: 1.0006x; 1.0006x over previous
import functools

import jax
import jax.numpy as jnp
from jax import lax
from jax.experimental import pallas as pl
from jax.experimental.pallas import tpu as pltpu
from jax.experimental.pallas import tpu_sc as plsc

B, S, H = 2, 2048, 1024
NH, NKV, HD = 16, 4, 64
I, E, K = 512, 8, 2
GRP = NH // NKV
EPS = 1e-6
T = B * S

BLK_T = 512
BLK_Q = 512
BLK_K = 512
QROWS = GRP * BLK_Q
NEG = -1e30

CAP = T
NBLK = 2 * T // BLK_T + E - 1
NW = 32
TPW = T // NW
HCH = TPW // 2


def _rms(x, w):
    var = jnp.mean(x * x, axis=-1, keepdims=True)
    return x * jax.lax.rsqrt(var + EPS) * w


def _pre_kernel(hs_ref, ln1_ref, wq_ref, wk_ref, wv_ref, q_ref, k_ref, v_ref):
    x = hs_ref[...]
    xn = _rms(x, ln1_ref[...]).astype(jnp.bfloat16)
    q = jax.lax.dot(xn, wq_ref[...],
                    preferred_element_type=jnp.float32).astype(jnp.bfloat16)
    k = jax.lax.dot(xn, wk_ref[...],
                    preferred_element_type=jnp.float32).astype(jnp.bfloat16)
    v = jax.lax.dot(xn, wv_ref[...],
                    preferred_element_type=jnp.float32).astype(jnp.bfloat16)
    for h in range(NH):
        q_ref[h] = q[:, h * HD:(h + 1) * HD]
    for h in range(NKV):
        k_ref[h] = k[:, h * HD:(h + 1) * HD]
        v_ref[h] = v[:, h * HD:(h + 1) * HD]


def _rope(x, cos, sin):
    x1 = x[:, :HD // 2]
    x2 = x[:, HD // 2:]
    rot = jnp.concatenate([-x2, x1], axis=-1)
    return x * cos + rot * sin


def _flash_kernel(iq_ref, ik_ref, q_ref, k_ref, v_ref, cq_ref, sq_ref,
                  ck_ref, sk_ref, mask_ref, o_ref, acc_ref, m_ref, l_ref):
    p_id = pl.program_id(2)
    iq = iq_ref[p_id]
    ik = ik_ref[p_id]

    @pl.when(ik == 0)
    def _():
        m_ref[...] = jnp.full_like(m_ref, NEG)
        l_ref[...] = jnp.zeros_like(l_ref)
        acc_ref[...] = jnp.zeros_like(acc_ref)

    q = _rope(q_ref[...].reshape(QROWS, HD).astype(jnp.float32),
              cq_ref[0], sq_ref[0])
    k = _rope(k_ref[0].astype(jnp.float32), ck_ref[...], sk_ref[...])
    qb = (q * (HD ** -0.5)).astype(jnp.bfloat16)
    kb = k.astype(jnp.bfloat16)
    s = jax.lax.dot_general(qb, kb, (((1,), (1,)), ((), ())),
                            preferred_element_type=jnp.float32)
    ind = (ik == iq).astype(jnp.float32)
    s = s + mask_ref[...] * ind

    m_prev = m_ref[...]
    l_prev = l_ref[...]
    m_cur = jnp.max(s, axis=1, keepdims=True)
    m_next = jnp.maximum(m_prev, m_cur)
    alpha = jnp.exp(m_prev - m_next)
    p = jnp.exp(s - m_next[:, :1])
    l_ref[...] = alpha * l_prev + jnp.sum(p, axis=1, keepdims=True)
    m_ref[...] = m_next
    pv = jax.lax.dot(p.astype(jnp.bfloat16), v_ref[0],
                     preferred_element_type=jnp.float32)
    acc_ref[...] = acc_ref[...] * alpha[:, :1] + pv

    @pl.when(ik == iq)
    def _():
        out = acc_ref[...] / l_ref[:, :1]
        o_ref[...] = out.reshape(GRP, BLK_Q, HD)


def _post_kernel(attn_ref, hs_ref, ln2_ref, wo_ref, rw_ref,
                 res2_ref, xn2_ref, wv_ref, pidx_ref, desc_ref, cnt_ref):
    i = pl.program_id(0)

    @pl.when(i == 0)
    def _():
        cnt_ref[...] = jnp.zeros_like(cnt_ref)

    a = jnp.concatenate([attn_ref[h] for h in range(NH)], axis=1)
    o = jax.lax.dot(a.astype(jnp.bfloat16), wo_ref[...],
                    preferred_element_type=jnp.float32)
    res2 = hs_ref[...] + o
    res2_ref[...] = res2
    xn = _rms(res2, ln2_ref[...])
    xnb = xn.astype(jnp.bfloat16)
    xn2_ref[...] = xnb.astype(jnp.float32)
    logits = jax.lax.dot(xnb, rw_ref[...], preferred_element_type=jnp.float32)
    lane = jax.lax.broadcasted_iota(jnp.int32, (BLK_T, 128), 1)
    valid = lane < E
    lg = jnp.where(valid, logits, NEG)
    mx = jnp.max(lg, axis=1, keepdims=True)
    ex = jnp.where(valid, jnp.exp(lg - mx), 0.0)
    denom = jnp.sum(ex, axis=1, keepdims=True)
    sc = ex / denom
    m1 = jnp.max(sc, axis=1, keepdims=True)
    idx1 = jnp.min(jnp.where(sc == m1, lane, 128), axis=1, keepdims=True)
    is1 = lane == idx1
    sc2 = jnp.where(is1, -1.0, sc)
    m2 = jnp.max(sc2, axis=1, keepdims=True)
    idx2 = jnp.min(jnp.where(sc2 == m2, lane, 128), axis=1, keepdims=True)
    is2 = lane == idx2
    wsum = m1 + m2
    wv_ref[...] = jnp.where(lane == 0, m1 / wsum,
                            jnp.where(lane == 1, m2 / wsum, 0.0))

    sel = (is1 | is2).astype(jnp.float32)
    rt = jax.lax.broadcasted_iota(jnp.int32, (BLK_T, BLK_T), 0)
    ct = jax.lax.broadcasted_iota(jnp.int32, (BLK_T, BLK_T), 1)
    tril = (ct < rt).astype(jnp.float32)
    prefix = jax.lax.dot(tril, sel, preferred_element_type=jnp.float32)
    dest = lane.astype(jnp.float32) * CAP + cnt_ref[...] + prefix
    p1f = jnp.sum(dest * is1.astype(jnp.float32), axis=1, keepdims=True)
    p2f = jnp.sum(dest * is2.astype(jnp.float32), axis=1, keepdims=True)
    pidx_ref[...] = jnp.where(lane == 0, p1f,
                              jnp.where(lane == 1, p2f, 0.0)).astype(jnp.int32)
    cnt_new = cnt_ref[...] + jnp.sum(sel, axis=0, keepdims=True)
    cnt_ref[...] = cnt_new

    @pl.when(i == T // BLK_T - 1)
    def _():
        rc = jax.lax.broadcasted_iota(jnp.int32, (128, 128), 0)
        cc = jax.lax.broadcasted_iota(jnp.int32, (128, 128), 1)
        amap = ((cc == rc // 8) & (rc < 64)).astype(jnp.float32)
        cntc = jax.lax.dot_general(amap, cnt_new, (((1,), (1,)), ((), ())),
                                   preferred_element_type=jnp.float32)
        ccol = jax.lax.broadcasted_iota(jnp.int32, (128, 1), 0)
        jcol = ccol % 8
        act = ((jcol.astype(jnp.float32) * BLK_T) < cntc) & (ccol < 64)
        actf = act.astype(jnp.float32)
        tri = (cc < rc).astype(jnp.float32)
        rank = jax.lax.dot(tri, actf, preferred_element_type=jnp.float32)
        srow = cc.astype(jnp.float32)
        smat = ((rank == srow) & act).astype(jnp.float32)
        ecolf = (ccol // 8).astype(jnp.float32)
        gcolf = ccol.astype(jnp.float32)
        blk_e = jax.lax.dot_general(ecolf, smat, (((0,), (0,)), ((), ())),
                                    preferred_element_type=jnp.float32)
        blk_g = jax.lax.dot_general(gcolf, smat, (((0,), (0,)), ((), ())),
                                    preferred_element_type=jnp.float32)
        zero6 = jnp.zeros((6, 128), jnp.float32)
        desc_ref[...] = jnp.concatenate([blk_e, blk_g, zero6],
                                        axis=0).astype(jnp.int32)


def _sc_mesh():
    return plsc.VectorSubcoreMesh(core_axis_name="c", subcore_axis_name="s")


def _sc_dispatch(xn2_3d, p1, p2):
    @functools.partial(
        pl.kernel,
        mesh=_sc_mesh(),
        out_type=jax.ShapeDtypeStruct((E * CAP, 8, 128), jnp.float32),
        scratch_types=[
            pltpu.VMEM((HCH,), jnp.int32),
            pltpu.VMEM((HCH,), jnp.int32),
            pltpu.VMEM((HCH, 8, 128), jnp.float32),
            pltpu.SemaphoreType.DMA,
            pltpu.SemaphoreType.DMA,
        ],
        compiler_params=pltpu.CompilerParams(use_tc_tiling_on_sc=True),
    )
    def k(xn2_hbm, p1_hbm, p2_hbm, xg_hbm, i1_v, i2_v, rows_v, sem1, sem2):
        wid = lax.axis_index("s") * 2 + lax.axis_index("c")
        base = wid * TPW
        for half in range(2):
            b2 = base + half * HCH
            pltpu.sync_copy(p1_hbm.at[pl.ds(b2, HCH)], i1_v)
            pltpu.sync_copy(p2_hbm.at[pl.ds(b2, HCH)], i2_v)
            pltpu.sync_copy(xn2_hbm.at[pl.ds(b2, HCH)], rows_v)
            c1 = pltpu.async_copy(rows_v, xg_hbm.at[i1_v], sem1)
            c2 = pltpu.async_copy(rows_v, xg_hbm.at[i2_v], sem2)
            c1.wait()
            c2.wait()

    return k(xn2_3d, p1, p2)


def _sc_collect(os_3d, p1, p2):
    @functools.partial(
        pl.kernel,
        mesh=_sc_mesh(),
        out_type=[
            jax.ShapeDtypeStruct((T, 8, 128), jnp.float32),
            jax.ShapeDtypeStruct((T, 8, 128), jnp.float32),
        ],
        scratch_types=[
            pltpu.VMEM((HCH,), jnp.int32),
            pltpu.VMEM((HCH, 8, 128), jnp.float32),
            pltpu.SemaphoreType.DMA,
        ],
        compiler_params=pltpu.CompilerParams(use_tc_tiling_on_sc=True),
    )
    def k(os_hbm, p1_hbm, p2_hbm, g1_hbm, g2_hbm, idx_v, rows_v, sem):
        wid = lax.axis_index("s") * 2 + lax.axis_index("c")
        base = wid * TPW
        for half in range(2):
            b2 = base + half * HCH
            pltpu.sync_copy(p1_hbm.at[pl.ds(b2, HCH)], idx_v)
            pltpu.async_copy(os_hbm.at[idx_v], rows_v, sem).wait()
            pltpu.sync_copy(rows_v, g1_hbm.at[pl.ds(b2, HCH)])
            pltpu.sync_copy(p2_hbm.at[pl.ds(b2, HCH)], idx_v)
            pltpu.async_copy(os_hbm.at[idx_v], rows_v, sem).wait()
            pltpu.sync_copy(rows_v, g2_hbm.at[pl.ds(b2, HCH)])

    return k(os_3d, p1, p2)


def _moe2_kernel(be_ref, bg_ref, x_ref, wg_ref, wu_ref, wd_ref, o_ref):
    x = x_ref[...].astype(jnp.bfloat16)
    gate = jax.lax.dot(x, wg_ref[0], preferred_element_type=jnp.float32)
    up = jax.lax.dot(x, wu_ref[0], preferred_element_type=jnp.float32)
    act = (gate * jax.lax.logistic(gate)) * up
    o_ref[...] = jax.lax.dot(act.astype(jnp.bfloat16), wd_ref[0],
                             preferred_element_type=jnp.float32)


def _comb_kernel(res2_ref, g1_ref, g2_ref, wv_ref, y_ref):
    wv = wv_ref[...]
    w1 = wv[:, :1]
    w2 = wv[:, 1:2]
    y_ref[...] = res2_ref[...] + w1 * g1_ref[...] + w2 * g2_ref[...]


def _build(hidden_states, cos, sin, Wq, Wk, Wv, Wo, ln1_w, ln2_w,
           router_w, Wg, Wu, Wd):
    hs = hidden_states.reshape(T, H)
    ln1 = ln1_w.reshape(1, H)
    ln2 = ln2_w.reshape(1, H)
    wq_t = Wq.T.astype(jnp.bfloat16)
    wk_t = Wk.T.astype(jnp.bfloat16)
    wv_t = Wv.T.astype(jnp.bfloat16)
    wo_t = Wo.T.astype(jnp.bfloat16)
    rw_pad = jnp.zeros((128, H), jnp.float32).at[:E].set(router_w)
    rw_t = rw_pad.T.astype(jnp.bfloat16)

    n_t = T // BLK_T
    q, k, v = pl.pallas_call(
        _pre_kernel,
        grid=(n_t,),
        in_specs=[
            pl.BlockSpec((BLK_T, H), lambda i: (i, 0)),
            pl.BlockSpec((1, H), lambda i: (0, 0)),
            pl.BlockSpec((H, NH * HD), lambda i: (0, 0)),
            pl.BlockSpec((H, NKV * HD), lambda i: (0, 0)),
            pl.BlockSpec((H, NKV * HD), lambda i: (0, 0)),
        ],
        out_specs=[
            pl.BlockSpec((NH, BLK_T, HD), lambda i: (0, i, 0)),
            pl.BlockSpec((NKV, BLK_T, HD), lambda i: (0, i, 0)),
            pl.BlockSpec((NKV, BLK_T, HD), lambda i: (0, i, 0)),
        ],
        out_shape=[
            jax.ShapeDtypeStruct((NH, T, HD), jnp.bfloat16),
            jax.ShapeDtypeStruct((NKV, T, HD), jnp.bfloat16),
            jax.ShapeDtypeStruct((NKV, T, HD), jnp.bfloat16),
        ],
    )(hs, ln1, wq_t, wk_t, wv_t)

    n_q = S // BLK_Q
    n_k = S // BLK_K
    cos_q = jnp.tile(cos.reshape(n_q, 1, BLK_Q, HD), (1, GRP, 1, 1)) \
        .reshape(n_q, QROWS, HD)
    sin_q = jnp.tile(sin.reshape(n_q, 1, BLK_Q, HD), (1, GRP, 1, 1)) \
        .reshape(n_q, QROWS, HD)
    r = jnp.arange(BLK_Q)[:, None]
    c = jnp.arange(BLK_K)[None, :]
    mask1 = jnp.where(r >= c, 0.0, NEG).astype(jnp.float32)
    mask = jnp.tile(mask1, (GRP, 1))

    pairs = [(a, b2) for a in range(n_q) for b2 in range(a + 1)]
    n_p = len(pairs)
    iq_arr = jnp.asarray([p[0] for p in pairs], jnp.int32)
    ik_arr = jnp.asarray([p[1] for p in pairs], jnp.int32)

    attn = pl.pallas_call(
        _flash_kernel,
        grid_spec=pltpu.PrefetchScalarGridSpec(
            num_scalar_prefetch=2,
            grid=(B, NKV, n_p),
            in_specs=[
                pl.BlockSpec((GRP, BLK_Q, HD),
                             lambda b, g, p, iqa, ika: (g, b * n_q + iqa[p], 0)),
                pl.BlockSpec((1, BLK_K, HD),
                             lambda b, g, p, iqa, ika: (g, b * n_k + ika[p], 0)),
                pl.BlockSpec((1, BLK_K, HD),
                             lambda b, g, p, iqa, ika: (g, b * n_k + ika[p], 0)),
                pl.BlockSpec((1, QROWS, HD),
                             lambda b, g, p, iqa, ika: (iqa[p], 0, 0)),
                pl.BlockSpec((1, QROWS, HD),
                             lambda b, g, p, iqa, ika: (iqa[p], 0, 0)),
                pl.BlockSpec((BLK_K, HD),
                             lambda b, g, p, iqa, ika: (ika[p], 0)),
                pl.BlockSpec((BLK_K, HD),
                             lambda b, g, p, iqa, ika: (ika[p], 0)),
                pl.BlockSpec((QROWS, BLK_K),
                             lambda b, g, p, iqa, ika: (0, 0)),
            ],
            out_specs=pl.BlockSpec(
                (GRP, BLK_Q, HD),
                lambda b, g, p, iqa, ika: (g, b * n_q + iqa[p], 0)),
            scratch_shapes=[
                pltpu.VMEM((QROWS, HD), jnp.float32),
                pltpu.VMEM((QROWS, 128), jnp.float32),
                pltpu.VMEM((QROWS, 128), jnp.float32),
            ],
        ),
        out_shape=jax.ShapeDtypeStruct((NH, T, HD), jnp.float32),
        compiler_params=pltpu.CompilerParams(
            dimension_semantics=("parallel", "parallel", "arbitrary"),
        ),
    )(iq_arr, ik_arr, q, k, v, cos_q, sin_q, cos, sin, mask)

    res2, xn2, wv, pidx, desc = pl.pallas_call(
        _post_kernel,
        grid=(n_t,),
        in_specs=[
            pl.BlockSpec((NH, BLK_T, HD), lambda i: (0, i, 0)),
            pl.BlockSpec((BLK_T, H), lambda i: (i, 0)),
            pl.BlockSpec((1, H), lambda i: (0, 0)),
            pl.BlockSpec((NH * HD, H), lambda i: (0, 0)),
            pl.BlockSpec((H, 128), lambda i: (0, 0)),
        ],
        out_specs=[
            pl.BlockSpec((BLK_T, H), lambda i: (i, 0)),
            pl.BlockSpec((BLK_T, H), lambda i: (i, 0)),
            pl.BlockSpec((BLK_T, 128), lambda i: (i, 0)),
            pl.BlockSpec((BLK_T, 128), lambda i: (i, 0)),
            pl.BlockSpec((8, 128), lambda i: (0, 0)),
        ],
        out_shape=[
            jax.ShapeDtypeStruct((T, H), jnp.float32),
            jax.ShapeDtypeStruct((T, H), jnp.float32),
            jax.ShapeDtypeStruct((T, 128), jnp.float32),
            jax.ShapeDtypeStruct((T, 128), jnp.int32),
            jax.ShapeDtypeStruct((8, 128), jnp.int32),
        ],
        scratch_shapes=[pltpu.VMEM((1, 128), jnp.float32)],
        compiler_params=pltpu.CompilerParams(
            dimension_semantics=("arbitrary",),
        ),
    )(attn, hs, ln2, wo_t, rw_t)

    p1 = pidx[:, 0]
    p2 = pidx[:, 1]
    be = desc[0]
    bg = desc[1]

    xg = _sc_dispatch(xn2.reshape(T, 8, 128), p1, p2)
    xg2 = xg.reshape(E * CAP, H)

    wg_t = Wg.transpose(0, 2, 1).astype(jnp.bfloat16)
    wu_t = Wu.transpose(0, 2, 1).astype(jnp.bfloat16)
    wd_t = Wd.transpose(0, 2, 1).astype(jnp.bfloat16)

    os_ = pl.pallas_call(
        _moe2_kernel,
        grid_spec=pltpu.PrefetchScalarGridSpec(
            num_scalar_prefetch=2,
            grid=(NBLK,),
            in_specs=[
                pl.BlockSpec((BLK_T, H), lambda s, be, bg: (bg[s], 0)),
                pl.BlockSpec((1, H, I), lambda s, be, bg: (be[s], 0, 0)),
                pl.BlockSpec((1, H, I), lambda s, be, bg: (be[s], 0, 0)),
                pl.BlockSpec((1, I, H), lambda s, be, bg: (be[s], 0, 0)),
            ],
            out_specs=pl.BlockSpec((BLK_T, H), lambda s, be, bg: (bg[s], 0)),
        ),
        out_shape=jax.ShapeDtypeStruct((E * CAP, H), jnp.float32),
        compiler_params=pltpu.CompilerParams(
            dimension_semantics=("arbitrary",),
        ),
    )(be, bg, xg2, wg_t, wu_t, wd_t)

    g1, g2 = _sc_collect(os_.reshape(E * CAP, 8, 128), p1, p2)

    y = pl.pallas_call(
        _comb_kernel,
        grid=(n_t,),
        in_specs=[
            pl.BlockSpec((BLK_T, H), lambda i: (i, 0)),
            pl.BlockSpec((BLK_T, H), lambda i: (i, 0)),
            pl.BlockSpec((BLK_T, H), lambda i: (i, 0)),
            pl.BlockSpec((BLK_T, 128), lambda i: (i, 0)),
        ],
        out_specs=pl.BlockSpec((BLK_T, H), lambda i: (i, 0)),
        out_shape=jax.ShapeDtypeStruct((T, H), jnp.float32),
    )(res2, g1.reshape(T, H), g2.reshape(T, H), wv)

    return y.reshape(B, S, H)


@jax.jit
def kernel(hidden_states, cos, sin, Wq, Wk, Wv, Wo, ln1_w, ln2_w,
           router_w, Wg, Wu, Wd):
    return _build(hidden_states, cos, sin, Wq, Wk, Wv, Wo, ln1_w, ln2_w,
                  router_w, Wg, Wu, Wd)


# speedup vs baseline: 1.3059x; 1.3052x over previous
import functools

import jax
import jax.numpy as jnp
from jax import lax
from jax.experimental import pallas as pl
from jax.experimental.pallas import tpu as pltpu
from jax.experimental.pallas import tpu_sc as plsc

B, S, H = 2, 2048, 1024
NH, NKV, HD = 16, 4, 64
I, E, K = 512, 8, 2
GRP = NH // NKV
EPS = 1e-6
T = B * S

BLK_T = 512
BLK_Q = 512
BLK_K = 512
QROWS = GRP * BLK_Q
NEG = -1e30

CAP = T
NBLK = 2 * T // BLK_T + E - 1
NW = 32
TPW = T // NW
HCH = TPW // 2


def _rms(x, w):
    var = jnp.mean(x * x, axis=-1, keepdims=True)
    return x * jax.lax.rsqrt(var + EPS) * w


def _pre_kernel(hs_ref, ln1_ref, wq_ref, wk_ref, wv_ref, q_ref, k_ref, v_ref):
    x = hs_ref[...]
    xn = _rms(x, ln1_ref[...]).astype(jnp.bfloat16)
    q = jax.lax.dot(xn, wq_ref[...],
                    preferred_element_type=jnp.float32).astype(jnp.bfloat16)
    k = jax.lax.dot(xn, wk_ref[...],
                    preferred_element_type=jnp.float32).astype(jnp.bfloat16)
    v = jax.lax.dot(xn, wv_ref[...],
                    preferred_element_type=jnp.float32).astype(jnp.bfloat16)
    for h in range(NH):
        q_ref[h] = q[:, h * HD:(h + 1) * HD]
    for h in range(NKV):
        k_ref[h] = k[:, h * HD:(h + 1) * HD]
        v_ref[h] = v[:, h * HD:(h + 1) * HD]


def _rope(x, cos, sin):
    x1 = x[:, :HD // 2]
    x2 = x[:, HD // 2:]
    rot = jnp.concatenate([-x2, x1], axis=-1)
    return x * cos + rot * sin


def _flash_kernel(iq_ref, ik_ref, q_ref, k_ref, v_ref, cq_ref, sq_ref,
                  ck_ref, sk_ref, mask_ref, o_ref, acc_ref, m_ref, l_ref):
    p_id = pl.program_id(2)
    iq = iq_ref[p_id]
    ik = ik_ref[p_id]

    @pl.when(ik == 0)
    def _():
        m_ref[...] = jnp.full_like(m_ref, NEG)
        l_ref[...] = jnp.zeros_like(l_ref)
        acc_ref[...] = jnp.zeros_like(acc_ref)

    q = _rope(q_ref[...].reshape(QROWS, HD).astype(jnp.float32),
              cq_ref[0], sq_ref[0])
    k = _rope(k_ref[0].astype(jnp.float32), ck_ref[...], sk_ref[...])
    qb = (q * (HD ** -0.5)).astype(jnp.bfloat16)
    kb = k.astype(jnp.bfloat16)
    s = jax.lax.dot_general(qb, kb, (((1,), (1,)), ((), ())),
                            preferred_element_type=jnp.float32)
    ind = (ik == iq).astype(jnp.float32)
    s = s + mask_ref[...] * ind

    m_prev = m_ref[...]
    l_prev = l_ref[...]
    m_cur = jnp.max(s, axis=1, keepdims=True)
    m_next = jnp.maximum(m_prev, m_cur)
    alpha = jnp.exp(m_prev - m_next)
    p = jnp.exp(s - m_next[:, :1])
    l_ref[...] = alpha * l_prev + jnp.sum(p, axis=1, keepdims=True)
    m_ref[...] = m_next
    pv = jax.lax.dot(p.astype(jnp.bfloat16), v_ref[0],
                     preferred_element_type=jnp.float32)
    acc_ref[...] = acc_ref[...] * alpha[:, :1] + pv

    @pl.when(ik == iq)
    def _():
        out = acc_ref[...] / l_ref[:, :1]
        o_ref[...] = out.reshape(GRP, BLK_Q, HD)


def _post_kernel(attn_ref, hs_ref, ln2_ref, wo_ref, rw_ref,
                 res2_ref, xn2_ref, wv_ref, pidx_ref, desc_ref, cnt_ref):
    i = pl.program_id(0)

    @pl.when(i == 0)
    def _():
        cnt_ref[...] = jnp.zeros_like(cnt_ref)

    a = jnp.concatenate([attn_ref[h] for h in range(NH)], axis=1)
    o = jax.lax.dot(a.astype(jnp.bfloat16), wo_ref[...],
                    preferred_element_type=jnp.float32)
    res2 = hs_ref[...] + o
    res2_ref[...] = res2
    xn = _rms(res2, ln2_ref[...])
    xnb = xn.astype(jnp.bfloat16)
    xq = xnb.astype(jnp.float32)
    for j in range(H // 128):
        xn2_ref[:, j, :] = xq[:, j * 128:(j + 1) * 128]
    logits = jax.lax.dot(xnb, rw_ref[...], preferred_element_type=jnp.float32)
    lane = jax.lax.broadcasted_iota(jnp.int32, (BLK_T, 128), 1)
    valid = lane < E
    lg = jnp.where(valid, logits, NEG)
    mx = jnp.max(lg, axis=1, keepdims=True)
    ex = jnp.where(valid, jnp.exp(lg - mx), 0.0)
    denom = jnp.sum(ex, axis=1, keepdims=True)
    sc = ex / denom
    m1 = jnp.max(sc, axis=1, keepdims=True)
    idx1 = jnp.min(jnp.where(sc == m1, lane, 128), axis=1, keepdims=True)
    is1 = lane == idx1
    sc2 = jnp.where(is1, -1.0, sc)
    m2 = jnp.max(sc2, axis=1, keepdims=True)
    idx2 = jnp.min(jnp.where(sc2 == m2, lane, 128), axis=1, keepdims=True)
    is2 = lane == idx2
    wsum = m1 + m2
    wv_ref[...] = jnp.where(lane == 0, m1 / wsum,
                            jnp.where(lane == 1, m2 / wsum, 0.0))

    sel = (is1 | is2).astype(jnp.float32)
    rt = jax.lax.broadcasted_iota(jnp.int32, (BLK_T, BLK_T), 0)
    ct = jax.lax.broadcasted_iota(jnp.int32, (BLK_T, BLK_T), 1)
    tril = (ct < rt).astype(jnp.float32)
    prefix = jax.lax.dot(tril, sel, preferred_element_type=jnp.float32)
    dest = lane.astype(jnp.float32) * CAP + cnt_ref[...] + prefix
    p1f = jnp.sum(dest * is1.astype(jnp.float32), axis=1, keepdims=True)
    p2f = jnp.sum(dest * is2.astype(jnp.float32), axis=1, keepdims=True)
    pidx_ref[...] = jnp.where(lane == 0, p1f,
                              jnp.where(lane == 1, p2f, 0.0)).astype(jnp.int32)
    cnt_new = cnt_ref[...] + jnp.sum(sel, axis=0, keepdims=True)
    cnt_ref[...] = cnt_new

    @pl.when(i == T // BLK_T - 1)
    def _():
        rc = jax.lax.broadcasted_iota(jnp.int32, (128, 128), 0)
        cc = jax.lax.broadcasted_iota(jnp.int32, (128, 128), 1)
        amap = ((cc == rc // 8) & (rc < 64)).astype(jnp.float32)
        cntc = jax.lax.dot_general(amap, cnt_new, (((1,), (1,)), ((), ())),
                                   preferred_element_type=jnp.float32)
        ccol = jax.lax.broadcasted_iota(jnp.int32, (128, 1), 0)
        jcol = ccol % 8
        act = ((jcol.astype(jnp.float32) * BLK_T) < cntc) & (ccol < 64)
        actf = act.astype(jnp.float32)
        tri = (cc < rc).astype(jnp.float32)
        rank = jax.lax.dot(tri, actf, preferred_element_type=jnp.float32)
        srow = cc.astype(jnp.float32)
        smat = ((rank == srow) & act).astype(jnp.float32)
        ecolf = (ccol // 8).astype(jnp.float32)
        gcolf = ccol.astype(jnp.float32)
        blk_e = jax.lax.dot_general(ecolf, smat, (((0,), (0,)), ((), ())),
                                    preferred_element_type=jnp.float32)
        blk_g = jax.lax.dot_general(gcolf, smat, (((0,), (0,)), ((), ())),
                                    preferred_element_type=jnp.float32)
        zero6 = jnp.zeros((6, 128), jnp.float32)
        desc_ref[...] = jnp.concatenate([blk_e, blk_g, zero6],
                                        axis=0).astype(jnp.int32)


def _sc_mesh():
    return plsc.VectorSubcoreMesh(core_axis_name="c", subcore_axis_name="s")


def _sc_dispatch(xn2_3d, p1, p2):
    @functools.partial(
        pl.kernel,
        mesh=_sc_mesh(),
        out_type=jax.ShapeDtypeStruct((E * CAP, 8, 128), jnp.float32),
        scratch_types=[
            pltpu.VMEM((HCH,), jnp.int32),
            pltpu.VMEM((HCH,), jnp.int32),
            pltpu.VMEM((HCH, 8, 128), jnp.float32),
            pltpu.SemaphoreType.DMA,
            pltpu.SemaphoreType.DMA,
        ],
        compiler_params=pltpu.CompilerParams(use_tc_tiling_on_sc=True),
    )
    def k(xn2_hbm, p1_hbm, p2_hbm, xg_hbm, i1_v, i2_v, rows_v, sem1, sem2):
        wid = lax.axis_index("s") * 2 + lax.axis_index("c")
        base = wid * TPW
        for half in range(2):
            b2 = base + half * HCH
            pltpu.sync_copy(p1_hbm.at[pl.ds(b2, HCH)], i1_v)
            pltpu.sync_copy(p2_hbm.at[pl.ds(b2, HCH)], i2_v)
            pltpu.sync_copy(xn2_hbm.at[pl.ds(b2, HCH)], rows_v)
            c1 = pltpu.async_copy(rows_v, xg_hbm.at[i1_v], sem1)
            c2 = pltpu.async_copy(rows_v, xg_hbm.at[i2_v], sem2)
            c1.wait()
            c2.wait()

    return k(xn2_3d, p1, p2)


def _sc_collect(os_3d, p1, p2):
    @functools.partial(
        pl.kernel,
        mesh=_sc_mesh(),
        out_type=[
            jax.ShapeDtypeStruct((T, 8, 128), jnp.float32),
            jax.ShapeDtypeStruct((T, 8, 128), jnp.float32),
        ],
        scratch_types=[
            pltpu.VMEM((HCH,), jnp.int32),
            pltpu.VMEM((HCH, 8, 128), jnp.float32),
            pltpu.SemaphoreType.DMA,
        ],
        compiler_params=pltpu.CompilerParams(use_tc_tiling_on_sc=True),
    )
    def k(os_hbm, p1_hbm, p2_hbm, g1_hbm, g2_hbm, idx_v, rows_v, sem):
        wid = lax.axis_index("s") * 2 + lax.axis_index("c")
        base = wid * TPW
        for half in range(2):
            b2 = base + half * HCH
            pltpu.sync_copy(p1_hbm.at[pl.ds(b2, HCH)], idx_v)
            pltpu.async_copy(os_hbm.at[idx_v], rows_v, sem).wait()
            pltpu.sync_copy(rows_v, g1_hbm.at[pl.ds(b2, HCH)])
            pltpu.sync_copy(p2_hbm.at[pl.ds(b2, HCH)], idx_v)
            pltpu.async_copy(os_hbm.at[idx_v], rows_v, sem).wait()
            pltpu.sync_copy(rows_v, g2_hbm.at[pl.ds(b2, HCH)])

    return k(os_3d, p1, p2)


def _moe2_kernel(be_ref, bg_ref, x_ref, wg_ref, wu_ref, wd_ref, o_ref):
    x = jnp.concatenate([x_ref[:, j, :] for j in range(H // 128)],
                        axis=1).astype(jnp.bfloat16)
    gate = jax.lax.dot(x, wg_ref[0], preferred_element_type=jnp.float32)
    up = jax.lax.dot(x, wu_ref[0], preferred_element_type=jnp.float32)
    act = (gate * jax.lax.logistic(gate)) * up
    down = jax.lax.dot(act.astype(jnp.bfloat16), wd_ref[0],
                       preferred_element_type=jnp.float32)
    for j in range(H // 128):
        o_ref[:, j, :] = down[:, j * 128:(j + 1) * 128]


def _comb_kernel(res2_ref, g1_ref, g2_ref, wv_ref, y_ref):
    wv = wv_ref[...]
    w1 = wv[:, :1]
    w2 = wv[:, 1:2]
    for j in range(H // 128):
        sl = slice(j * 128, (j + 1) * 128)
        y_ref[:, sl] = (res2_ref[:, sl] + w1 * g1_ref[:, j, :]
                        + w2 * g2_ref[:, j, :])


def _build(hidden_states, cos, sin, Wq, Wk, Wv, Wo, ln1_w, ln2_w,
           router_w, Wg, Wu, Wd):
    hs = hidden_states.reshape(T, H)
    ln1 = ln1_w.reshape(1, H)
    ln2 = ln2_w.reshape(1, H)
    wq_t = Wq.T.astype(jnp.bfloat16)
    wk_t = Wk.T.astype(jnp.bfloat16)
    wv_t = Wv.T.astype(jnp.bfloat16)
    wo_t = Wo.T.astype(jnp.bfloat16)
    rw_pad = jnp.zeros((128, H), jnp.float32).at[:E].set(router_w)
    rw_t = rw_pad.T.astype(jnp.bfloat16)

    n_t = T // BLK_T
    q, k, v = pl.pallas_call(
        _pre_kernel,
        grid=(n_t,),
        in_specs=[
            pl.BlockSpec((BLK_T, H), lambda i: (i, 0)),
            pl.BlockSpec((1, H), lambda i: (0, 0)),
            pl.BlockSpec((H, NH * HD), lambda i: (0, 0)),
            pl.BlockSpec((H, NKV * HD), lambda i: (0, 0)),
            pl.BlockSpec((H, NKV * HD), lambda i: (0, 0)),
        ],
        out_specs=[
            pl.BlockSpec((NH, BLK_T, HD), lambda i: (0, i, 0)),
            pl.BlockSpec((NKV, BLK_T, HD), lambda i: (0, i, 0)),
            pl.BlockSpec((NKV, BLK_T, HD), lambda i: (0, i, 0)),
        ],
        out_shape=[
            jax.ShapeDtypeStruct((NH, T, HD), jnp.bfloat16),
            jax.ShapeDtypeStruct((NKV, T, HD), jnp.bfloat16),
            jax.ShapeDtypeStruct((NKV, T, HD), jnp.bfloat16),
        ],
    )(hs, ln1, wq_t, wk_t, wv_t)

    n_q = S // BLK_Q
    n_k = S // BLK_K
    cos_q = jnp.tile(cos.reshape(n_q, 1, BLK_Q, HD), (1, GRP, 1, 1)) \
        .reshape(n_q, QROWS, HD)
    sin_q = jnp.tile(sin.reshape(n_q, 1, BLK_Q, HD), (1, GRP, 1, 1)) \
        .reshape(n_q, QROWS, HD)
    r = jnp.arange(BLK_Q)[:, None]
    c = jnp.arange(BLK_K)[None, :]
    mask1 = jnp.where(r >= c, 0.0, NEG).astype(jnp.float32)
    mask = jnp.tile(mask1, (GRP, 1))

    pairs = [(a, b2) for a in range(n_q) for b2 in range(a + 1)]
    n_p = len(pairs)
    iq_arr = jnp.asarray([p[0] for p in pairs], jnp.int32)
    ik_arr = jnp.asarray([p[1] for p in pairs], jnp.int32)

    attn = pl.pallas_call(
        _flash_kernel,
        grid_spec=pltpu.PrefetchScalarGridSpec(
            num_scalar_prefetch=2,
            grid=(B, NKV, n_p),
            in_specs=[
                pl.BlockSpec((GRP, BLK_Q, HD),
                             lambda b, g, p, iqa, ika: (g, b * n_q + iqa[p], 0)),
                pl.BlockSpec((1, BLK_K, HD),
                             lambda b, g, p, iqa, ika: (g, b * n_k + ika[p], 0)),
                pl.BlockSpec((1, BLK_K, HD),
                             lambda b, g, p, iqa, ika: (g, b * n_k + ika[p], 0)),
                pl.BlockSpec((1, QROWS, HD),
                             lambda b, g, p, iqa, ika: (iqa[p], 0, 0)),
                pl.BlockSpec((1, QROWS, HD),
                             lambda b, g, p, iqa, ika: (iqa[p], 0, 0)),
                pl.BlockSpec((BLK_K, HD),
                             lambda b, g, p, iqa, ika: (ika[p], 0)),
                pl.BlockSpec((BLK_K, HD),
                             lambda b, g, p, iqa, ika: (ika[p], 0)),
                pl.BlockSpec((QROWS, BLK_K),
                             lambda b, g, p, iqa, ika: (0, 0)),
            ],
            out_specs=pl.BlockSpec(
                (GRP, BLK_Q, HD),
                lambda b, g, p, iqa, ika: (g, b * n_q + iqa[p], 0)),
            scratch_shapes=[
                pltpu.VMEM((QROWS, HD), jnp.float32),
                pltpu.VMEM((QROWS, 128), jnp.float32),
                pltpu.VMEM((QROWS, 128), jnp.float32),
            ],
        ),
        out_shape=jax.ShapeDtypeStruct((NH, T, HD), jnp.float32),
        compiler_params=pltpu.CompilerParams(
            dimension_semantics=("parallel", "parallel", "arbitrary"),
        ),
    )(iq_arr, ik_arr, q, k, v, cos_q, sin_q, cos, sin, mask)

    res2, xn2, wv, pidx, desc = pl.pallas_call(
        _post_kernel,
        grid=(n_t,),
        in_specs=[
            pl.BlockSpec((NH, BLK_T, HD), lambda i: (0, i, 0)),
            pl.BlockSpec((BLK_T, H), lambda i: (i, 0)),
            pl.BlockSpec((1, H), lambda i: (0, 0)),
            pl.BlockSpec((NH * HD, H), lambda i: (0, 0)),
            pl.BlockSpec((H, 128), lambda i: (0, 0)),
        ],
        out_specs=[
            pl.BlockSpec((BLK_T, H), lambda i: (i, 0)),
            pl.BlockSpec((BLK_T, H // 128, 128), lambda i: (i, 0, 0)),
            pl.BlockSpec((BLK_T, 128), lambda i: (i, 0)),
            pl.BlockSpec((BLK_T, 128), lambda i: (i, 0)),
            pl.BlockSpec((8, 128), lambda i: (0, 0)),
        ],
        out_shape=[
            jax.ShapeDtypeStruct((T, H), jnp.float32),
            jax.ShapeDtypeStruct((T, H // 128, 128), jnp.float32),
            jax.ShapeDtypeStruct((T, 128), jnp.float32),
            jax.ShapeDtypeStruct((T, 128), jnp.int32),
            jax.ShapeDtypeStruct((8, 128), jnp.int32),
        ],
        scratch_shapes=[pltpu.VMEM((1, 128), jnp.float32)],
        compiler_params=pltpu.CompilerParams(
            dimension_semantics=("arbitrary",),
        ),
    )(attn, hs, ln2, wo_t, rw_t)

    p1 = pidx[:, 0]
    p2 = pidx[:, 1]
    be = desc[0]
    bg = desc[1]

    xg = _sc_dispatch(xn2, p1, p2)

    wg_t = Wg.transpose(0, 2, 1).astype(jnp.bfloat16)
    wu_t = Wu.transpose(0, 2, 1).astype(jnp.bfloat16)
    wd_t = Wd.transpose(0, 2, 1).astype(jnp.bfloat16)

    os_ = pl.pallas_call(
        _moe2_kernel,
        grid_spec=pltpu.PrefetchScalarGridSpec(
            num_scalar_prefetch=2,
            grid=(NBLK,),
            in_specs=[
                pl.BlockSpec((BLK_T, H // 128, 128),
                             lambda s, be, bg: (bg[s], 0, 0)),
                pl.BlockSpec((1, H, I), lambda s, be, bg: (be[s], 0, 0)),
                pl.BlockSpec((1, H, I), lambda s, be, bg: (be[s], 0, 0)),
                pl.BlockSpec((1, I, H), lambda s, be, bg: (be[s], 0, 0)),
            ],
            out_specs=pl.BlockSpec((BLK_T, H // 128, 128),
                                   lambda s, be, bg: (bg[s], 0, 0)),
        ),
        out_shape=jax.ShapeDtypeStruct((E * CAP, H // 128, 128), jnp.float32),
        compiler_params=pltpu.CompilerParams(
            dimension_semantics=("arbitrary",),
        ),
    )(be, bg, xg, wg_t, wu_t, wd_t)

    g1, g2 = _sc_collect(os_, p1, p2)

    y = pl.pallas_call(
        _comb_kernel,
        grid=(n_t,),
        in_specs=[
            pl.BlockSpec((BLK_T, H), lambda i: (i, 0)),
            pl.BlockSpec((BLK_T, H // 128, 128), lambda i: (i, 0, 0)),
            pl.BlockSpec((BLK_T, H // 128, 128), lambda i: (i, 0, 0)),
            pl.BlockSpec((BLK_T, 128), lambda i: (i, 0)),
        ],
        out_specs=pl.BlockSpec((BLK_T, H), lambda i: (i, 0)),
        out_shape=jax.ShapeDtypeStruct((T, H), jnp.float32),
    )(res2, g1, g2, wv)

    return y.reshape(B, S, H)


@jax.jit
def kernel(hidden_states, cos, sin, Wq, Wk, Wv, Wo, ln1_w, ln2_w,
           router_w, Wg, Wu, Wd):
    return _build(hidden_states, cos, sin, Wq, Wk, Wv, Wo, ln1_w, ln2_w,
                  router_w, Wg, Wu, Wd)
